# baseline (speedup 1.0000x reference)
import jax
import jax.numpy as jnp
from jax import lax
from jax.experimental import pallas as pl
from jax.experimental.pallas import tpu as pltpu
from jax.experimental.pallas import tpu_sc as plsc

_NUM_CLASSES = 1000
_BATCH = 512
_NC = 2
_NS = 16
_NW = _NC * _NS
_SPW = _BATCH // _NW
_LANES = 16
_CHUNKS = _SPW * 24
_ROW_ELEMS = 16
_SLAB = _SPW * _NUM_CLASSES


def _body(x_hbm, out_hbm, idx_v, rows_v, ov, sem):
    wid = lax.axis_index("s") * _NC + lax.axis_index("c")
    base = wid * _SPW

    lanes = lax.iota(jnp.int32, _LANES)

    for v in range(_CHUNKS // _LANES):
        j = lanes + (v * _LANES)
        s = j // 24
        t = j - s * 24
        c = t // 8
        r = t - c * 8
        row = (base + s) * 9408 + c * 3136 + r * 14
        idx_v[v // 8, pl.ds((v % 8) * _LANES, _LANES)] = row

    copies = [
        pltpu.async_copy(
            x_hbm.at[idx_v.at[k]], rows_v.at[pl.ds(k * 128, 128)], sem
        )
        for k in range(3)
    ]

    zeros16 = jnp.zeros((_LANES,), jnp.float32)

    def _zero(i, carry):
        off = pl.multiple_of(i * _LANES, _LANES)
        ov[pl.ds(off, _LANES)] = zeros16
        return carry

    lax.fori_loop(0, _SLAB // _LANES, _zero, 0)

    for cp in copies:
        cp.wait()

    acc = jnp.zeros((_LANES,), jnp.float32)
    row_base = lanes * 24
    for t in range(24):
        rvec = row_base + t
        for col in range(8):
            cvec = jnp.full((_LANES,), col, jnp.int32)
            acc = acc + plsc.load_gather(rows_v, [rvec, cvec])

    mean = acc / 192.0
    scaled = mean * 10.0
    pred = scaled.astype(jnp.int32)
    rem = lax.rem(pred, _NUM_CLASSES)
    pred = jnp.where(rem < 0, rem + _NUM_CLASSES, rem)

    flat = lanes * _NUM_CLASSES + pred
    plsc.store_scatter(ov, [flat], jnp.full((_LANES,), 10.0, jnp.float32))

    pltpu.sync_copy(ov, out_hbm.at[pl.ds(wid * _SLAB, _SLAB)])


@jax.jit
def kernel(x):
    xr = x.reshape(-1, _ROW_ELEMS)
    out = pl.kernel(
        _body,
        out_type=jax.ShapeDtypeStruct((_BATCH * _NUM_CLASSES,), jnp.float32),
        mesh=plsc.VectorSubcoreMesh(core_axis_name="c", subcore_axis_name="s"),
        scratch_types=[
            pltpu.VMEM((3, 128), jnp.int32),
            pltpu.VMEM((_CHUNKS, _ROW_ELEMS), jnp.float32),
            pltpu.VMEM((_SLAB,), jnp.float32),
            pltpu.SemaphoreType.DMA,
        ],
        compiler_params=pltpu.CompilerParams(
            needs_layout_passes=False, use_tc_tiling_on_sc=False
        ),
    )(xr)
    return out.reshape(_BATCH, _NUM_CLASSES)


# speedup vs baseline: 2.0655x; 2.0655x over previous
import jax
import jax.numpy as jnp
from jax import lax
from jax.experimental import pallas as pl
from jax.experimental.pallas import tpu as pltpu
from jax.experimental.pallas import tpu_sc as plsc

_NUM_CLASSES = 1000
_BATCH = 512
_NC = 2
_NS = 16
_NW = _NC * _NS
_SPW = _BATCH // _NW
_LANES = 16


def _body(x_hbm, out_hbm, xv, ov, sem):
    wid = lax.axis_index("s") * _NC + lax.axis_index("c")
    base = wid * _SPW

    cp = pltpu.async_copy(
        x_hbm.at[pl.ds(base, _SPW), :, pl.ds(0, 8), pl.ds(0, 128)], xv, sem
    )

    zeros16 = jnp.zeros((_LANES,), jnp.float32)
    for srow in range(_SPW):

        def _zero(i, carry, srow=srow):
            off = pl.multiple_of(i * _LANES, _LANES)
            ov[srow, pl.ds(off, _LANES)] = zeros16
            return carry

        lax.fori_loop(0, _NUM_CLASSES // _LANES, _zero, 0)
        ov[srow, pl.ds(_NUM_CLASSES - _LANES, _LANES)] = zeros16

    cp.wait()

    lanes = lax.iota(jnp.int32, _LANES)
    acc = jnp.zeros((_LANES,), jnp.float32)
    for c in range(3):
        cvec = jnp.full((_LANES,), c, jnp.int32)
        for r in range(8):
            rvec = jnp.full((_LANES,), r, jnp.int32)
            for col in range(8):
                colvec = jnp.full((_LANES,), col, jnp.int32)
                acc = acc + plsc.load_gather(xv, [lanes, cvec, rvec, colvec])

    mean = acc / 192.0
    scaled = mean * 10.0
    pred = scaled.astype(jnp.int32)
    rem = lax.rem(pred, _NUM_CLASSES)
    pred = jnp.where(rem < 0, rem + _NUM_CLASSES, rem)

    plsc.store_scatter(ov, [lanes, pred], jnp.full((_LANES,), 10.0, jnp.float32))

    pltpu.sync_copy(ov, out_hbm.at[pl.ds(base, _SPW), :])


@jax.jit
def kernel(x):
    return pl.kernel(
        _body,
        out_type=jax.ShapeDtypeStruct((_BATCH, _NUM_CLASSES), jnp.float32),
        mesh=plsc.VectorSubcoreMesh(core_axis_name="c", subcore_axis_name="s"),
        scratch_types=[
            pltpu.VMEM((_SPW, 3, 8, 128), jnp.float32),
            pltpu.VMEM((_SPW, _NUM_CLASSES), jnp.float32),
            pltpu.SemaphoreType.DMA,
        ],
        compiler_params=pltpu.CompilerParams(needs_layout_passes=False),
    )(x)


# speedup vs baseline: 24.9702x; 12.0890x over previous
import jax
import jax.numpy as jnp
from jax import lax
from jax.experimental import pallas as pl
from jax.experimental.pallas import tpu as pltpu
from jax.experimental.pallas import tpu_sc as plsc

_NUM_CLASSES = 1000
_BATCH = 512
_LANES = 16


def _body(xt_hbm, out_hbm, xv, ov, pbuf, p8, s16, shared, sem):
    c = lax.axis_index("c")
    s = lax.axis_index("s")
    g = s // 8
    h = s - g * 8
    bj = 2 * c + g
    b0 = bj * 128

    cp = pltpu.async_copy(
        xt_hbm.at[:, h, pl.ds(0, 8), pl.ds(b0, 128)], xv, sem
    )

    zeros16 = jnp.zeros((_LANES,), jnp.float32)
    for srow in range(16):

        def _zero(i, carry, srow=srow):
            off = pl.multiple_of(i * _LANES, _LANES)
            ov[srow, pl.ds(off, _LANES)] = zeros16
            return carry

        lax.fori_loop(0, _NUM_CLASSES // _LANES, _zero, 0)
        ov[srow, pl.ds(_NUM_CLASSES - _LANES, _LANES)] = zeros16

    cp.wait()

    for k in range(8):
        acc = jnp.zeros((_LANES,), jnp.float32)
        for ch in range(3):
            for w in range(8):
                acc = acc + xv[ch, w, pl.ds(k * _LANES, _LANES)]
        pbuf[pl.ds(k * _LANES, _LANES)] = acc

    pltpu.sync_copy(pbuf, shared.at[s])
    plsc.subcore_barrier()
    pltpu.sync_copy(shared.at[pl.ds(g * 8, 8), :], p8)

    hoff = pl.multiple_of(h * _LANES, _LANES)
    total = jnp.zeros((_LANES,), jnp.float32)
    for i in range(8):
        total = total + p8[i, pl.ds(hoff, _LANES)]

    mean = total / 192.0
    scaled = mean * 10.0
    pred = scaled.astype(jnp.int32)
    rem = lax.rem(pred, _NUM_CLASSES)
    pred = jnp.where(rem < 0, rem + _NUM_CLASSES, rem)

    lanes = lax.iota(jnp.int32, _LANES)
    plsc.store_scatter(ov, [lanes, pred], jnp.full((_LANES,), 10.0, jnp.float32))

    base = b0 + h * _LANES
    pltpu.sync_copy(ov, out_hbm.at[pl.ds(base, _LANES), :])
    del s16


@jax.jit
def kernel(x):
    xt = jnp.transpose(x, (1, 2, 3, 0))
    return pl.kernel(
        _body,
        out_type=jax.ShapeDtypeStruct((_BATCH, _NUM_CLASSES), jnp.float32),
        mesh=plsc.VectorSubcoreMesh(core_axis_name="c", subcore_axis_name="s"),
        scratch_types=[
            pltpu.VMEM((3, 8, 128), jnp.float32),
            pltpu.VMEM((_LANES, _NUM_CLASSES), jnp.float32),
            pltpu.VMEM((128,), jnp.float32),
            pltpu.VMEM((8, 128), jnp.float32),
            pltpu.VMEM((_LANES,), jnp.float32),
            pltpu.VMEM_SHARED((16, 128), jnp.float32),
            pltpu.SemaphoreType.DMA,
        ],
        compiler_params=pltpu.CompilerParams(needs_layout_passes=False),
    )(xt)
